# Initial kernel scaffold; baseline (speedup 1.0000x reference)
import jax
import jax.numpy as jnp
from jax.experimental import pallas as pl


def kernel(sparse_0, sparse_1, sparse_2, sparse_3, sparse_4, sparse_5, sparse_6, sparse_7, sparse_8, sparse_9, sparse_10, sparse_11, sparse_12, sparse_13, sparse_14, sparse_15, sparse_16, sparse_17, sparse_18, sparse_19, sparse_20, sparse_21, sparse_22, sparse_23, sparse_24, sparse_25, hist, W_0, W_1, W_2, W_3, W_4, W_5, W_6, W_7, W_8, W_9, W_10, W_11, W_12, W_13, W_14, W_15, W_16, W_17, W_18, W_19, W_20, W_21, W_22, W_23, W_24, W_25, W_hist):
    raise NotImplementedError("write your pallas kernel here")


# speedup vs baseline: 1.1560x; 1.1560x over previous
import functools

import jax
import jax.numpy as jnp
from jax import lax
from jax.experimental import pallas as pl
from jax.experimental.pallas import tpu as pltpu
from jax.experimental.pallas import tpu_sc as plsc

N_SPARSE = 26
VOCAB = 100000
DIM = 32
BATCH = 4096
HIST_LEN = 50

NUM_CORES = 2
NUM_SUBCORES = 16
NUM_WORKERS = NUM_CORES * NUM_SUBCORES
ROWS_PER_WORKER = BATCH // NUM_WORKERS
OUT_COLS = N_SPARSE * DIM + HIST_LEN


def _row_mean_body(w_ref, o_ref):
    o_ref[...] = jnp.sum(w_ref[...], axis=1, keepdims=True) * (1.0 / DIM)


def _row_mean(w_hist):
    blk = 10000
    out = pl.pallas_call(
        _row_mean_body,
        grid=(VOCAB // blk,),
        in_specs=[pl.BlockSpec((blk, DIM), lambda i: (i, 0))],
        out_specs=pl.BlockSpec((blk, 1), lambda i: (i, 0)),
        out_shape=jax.ShapeDtypeStruct((VOCAB, 1), jnp.float32),
    )(w_hist)
    return out.reshape(VOCAB)


def _sc_body(*refs):
    tables = refs[:N_SPARSE]
    sidx_ref = refs[N_SPARSE]
    hist_ref = refs[N_SPARSE + 1]
    rm_ref = refs[N_SPARSE + 2]
    out_ref = refs[N_SPARSE + 3]
    hout_ref = refs[N_SPARSE + 4]
    idx_v, rows_v, hidx_v, hvals_v, sem = refs[N_SPARSE + 5:]

    c = lax.axis_index("c")
    s = lax.axis_index("s")
    wid = s * NUM_CORES + c
    base = wid * ROWS_PER_WORKER

    for i in range(N_SPARSE):
        pltpu.sync_copy(sidx_ref.at[i, pl.ds(base, ROWS_PER_WORKER)], idx_v)
        pltpu.async_copy(tables[i].at[idx_v], rows_v, sem).wait()
        pltpu.sync_copy(
            rows_v,
            out_ref.at[pl.ds(base, ROWS_PER_WORKER), pl.ds(i * DIM, DIM)],
        )

    nh = ROWS_PER_WORKER * HIST_LEN
    pltpu.sync_copy(hist_ref.at[pl.ds(base * HIST_LEN, nh)], hidx_v)
    pltpu.async_copy(rm_ref.at[hidx_v], hvals_v, sem).wait()
    pltpu.sync_copy(hvals_v, hout_ref.at[pl.ds(base * HIST_LEN, nh)])


def kernel(sparse_0, sparse_1, sparse_2, sparse_3, sparse_4, sparse_5,
           sparse_6, sparse_7, sparse_8, sparse_9, sparse_10, sparse_11,
           sparse_12, sparse_13, sparse_14, sparse_15, sparse_16, sparse_17,
           sparse_18, sparse_19, sparse_20, sparse_21, sparse_22, sparse_23,
           sparse_24, sparse_25, hist,
           W_0, W_1, W_2, W_3, W_4, W_5, W_6, W_7, W_8, W_9, W_10, W_11,
           W_12, W_13, W_14, W_15, W_16, W_17, W_18, W_19, W_20, W_21,
           W_22, W_23, W_24, W_25, W_hist):
    sparse = [sparse_0, sparse_1, sparse_2, sparse_3, sparse_4, sparse_5,
              sparse_6, sparse_7, sparse_8, sparse_9, sparse_10, sparse_11,
              sparse_12, sparse_13, sparse_14, sparse_15, sparse_16,
              sparse_17, sparse_18, sparse_19, sparse_20, sparse_21,
              sparse_22, sparse_23, sparse_24, sparse_25]
    tables = [W_0, W_1, W_2, W_3, W_4, W_5, W_6, W_7, W_8, W_9, W_10, W_11,
              W_12, W_13, W_14, W_15, W_16, W_17, W_18, W_19, W_20, W_21,
              W_22, W_23, W_24, W_25]

    sidx = jnp.concatenate([x.reshape(1, BATCH) for x in sparse], axis=0)
    rm = _row_mean(W_hist)

    mesh = plsc.VectorSubcoreMesh(core_axis_name="c", subcore_axis_name="s")
    sc = pl.kernel(
        _sc_body,
        out_type=(
            jax.ShapeDtypeStruct((BATCH, N_SPARSE * DIM), jnp.float32),
            jax.ShapeDtypeStruct((BATCH * HIST_LEN,), jnp.float32),
        ),
        mesh=mesh,
        compiler_params=pltpu.CompilerParams(use_tc_tiling_on_sc=False),
        scratch_types=[
            pltpu.VMEM((ROWS_PER_WORKER,), jnp.int32),
            pltpu.VMEM((ROWS_PER_WORKER, DIM), jnp.float32),
            pltpu.VMEM((ROWS_PER_WORKER * HIST_LEN,), jnp.int32),
            pltpu.VMEM((ROWS_PER_WORKER * HIST_LEN,), jnp.float32),
            pltpu.SemaphoreType.DMA,
        ],
    )
    out_sparse, out_hist = sc(*tables, sidx, hist.reshape(-1), rm)
    return jnp.concatenate(
        [out_sparse, out_hist.reshape(BATCH, HIST_LEN)], axis=-1)
